# baseline (speedup 1.0000x reference)
import functools

import jax
import jax.numpy as jnp
from jax.experimental import pallas as pl


def _som_tile(zt_ref, cb_ref, zsq_ref, csq_ref, iota_ref, zqt_ref,
              q_ref, q2_ref, bmu_ref, *, alpha, k):
    z = jnp.swapaxes(zt_ref[0], 0, 1)
    cb = cb_ref[...]
    z_sq = jnp.swapaxes(zsq_ref[0], 0, 1)
    c_sq = csq_ref[...]

    cross = jax.lax.dot_general(z, cb, (((1,), (1,)), ((), ())),
                                preferred_element_type=jnp.float32)
    cross2 = cross * (-2.0)
    d = jnp.maximum(z_sq + cross2 + c_sq, 0.0)

    t = 1.0 + d * (1.0 / alpha)
    r = pl.reciprocal(t, approx=True)
    u = r * r * r
    s = jnp.sum(u, axis=1, keepdims=True)
    q = u * pl.reciprocal(s, approx=True)
    q_ref[...] = q
    q2_ref[...] = q

    d_min = jnp.min(d, axis=1, keepdims=True)
    iota_f = iota_ref[...]
    masked = jnp.where(d == d_min, iota_f, float(2 * k))
    bmu_f = jnp.min(masked, axis=1, keepdims=True)
    bmu_ref[...] = bmu_f[:, 0].astype(jnp.int32)

    one_hot = (iota_f == bmu_f).astype(jnp.float32)
    zq = jnp.dot(one_hot, cb, preferred_element_type=jnp.float32,
                 precision=jax.lax.Precision.HIGHEST)
    zqt_ref[...] = jnp.swapaxes(zq, 0, 1)


def kernel(ts_emb_seq, codebook):
    alpha = 5.0
    b, t_max, d_latent = ts_emb_seq.shape
    n = b * t_max
    k = codebook.shape[0]
    z = ts_emb_seq.reshape(n, d_latent)
    zt = jnp.transpose(ts_emb_seq, (0, 2, 1))
    z_sq = jnp.sum(z * z, axis=1, keepdims=True)
    zsq3 = z_sq.reshape(b, 1, t_max)
    c_sq = jnp.sum(codebook * codebook, axis=1)[None, :]
    iota_f = jnp.arange(k, dtype=jnp.float32)[None, :]

    tn = t_max
    grid = (b,)

    zqt, q, q2, bmu = pl.pallas_call(
        functools.partial(_som_tile, alpha=alpha, k=k),
        grid=grid,
        in_specs=[
            pl.BlockSpec((1, d_latent, tn), lambda i: (i, 0, 0)),
            pl.BlockSpec((k, d_latent), lambda i: (0, 0)),
            pl.BlockSpec((1, 1, tn), lambda i: (i, 0, 0)),
            pl.BlockSpec((1, k), lambda i: (0, 0)),
            pl.BlockSpec((1, k), lambda i: (0, 0)),
        ],
        out_specs=[
            pl.BlockSpec((d_latent, tn), lambda i: (0, i)),
            pl.BlockSpec((tn, k), lambda i: (i, 0)),
            pl.BlockSpec((tn, k), lambda i: (i, 0)),
            pl.BlockSpec((tn,), lambda i: (i,)),
        ],
        out_shape=[
            jax.ShapeDtypeStruct((d_latent, n), jnp.float32),
            jax.ShapeDtypeStruct((n, k), jnp.float32),
            jax.ShapeDtypeStruct((n, k), jnp.float32),
            jax.ShapeDtypeStruct((n,), jnp.int32),
        ],
    )(zt, codebook, zsq3, c_sq, iota_f)

    return zqt.T, q, q2, bmu


# speedup vs baseline: 1.5825x; 1.5825x over previous
import functools

import jax
import jax.numpy as jnp
from jax.experimental import pallas as pl


def _som_tile(zt_ref, cb_ref, zsq_ref, csq_ref, iota_ref, zqt_ref,
              q_ref, q2_ref, bmu_ref, *, alpha, k):
    z = jnp.swapaxes(zt_ref[0], 0, 1)
    cb = cb_ref[...]
    z_sq = jnp.swapaxes(zsq_ref[0], 0, 1)
    c_sq = csq_ref[...]

    cross = jax.lax.dot_general(z, cb, (((1,), (1,)), ((), ())),
                                preferred_element_type=jnp.float32)
    cross2 = cross * (-2.0)
    d = jnp.maximum(z_sq + cross2 + c_sq, 0.0)

    t = 1.0 + d * (1.0 / alpha)
    r = pl.reciprocal(t, approx=True)
    u = r * r * r
    s = jnp.sum(u, axis=1, keepdims=True)
    q = u * pl.reciprocal(s, approx=True)
    q_ref[...] = q
    q2_ref[...] = q

    d_min = jnp.min(d, axis=1, keepdims=True)
    iota_f = iota_ref[...]
    masked = jnp.where(d == d_min, iota_f, float(2 * k))
    bmu_f = jnp.min(masked, axis=1, keepdims=True)
    bmu_ref[...] = bmu_f[:, 0].astype(jnp.int32)

    one_hot = (iota_f == bmu_f).astype(jnp.float32)
    zq = jnp.dot(one_hot, cb, preferred_element_type=jnp.float32)
    zqt_ref[...] = jnp.swapaxes(zq, 0, 1)


def kernel(ts_emb_seq, codebook):
    alpha = 5.0
    b, t_max, d_latent = ts_emb_seq.shape
    n = b * t_max
    k = codebook.shape[0]
    z = ts_emb_seq.reshape(n, d_latent)
    zt = jnp.transpose(ts_emb_seq, (0, 2, 1))
    z_sq = jnp.sum(z * z, axis=1, keepdims=True)
    zsq3 = z_sq.reshape(b, 1, t_max)
    c_sq = jnp.sum(codebook * codebook, axis=1)[None, :]
    iota_f = jnp.arange(k, dtype=jnp.float32)[None, :]

    tn = t_max
    grid = (b,)

    zqt, q, q2, bmu = pl.pallas_call(
        functools.partial(_som_tile, alpha=alpha, k=k),
        grid=grid,
        in_specs=[
            pl.BlockSpec((1, d_latent, tn), lambda i: (i, 0, 0)),
            pl.BlockSpec((k, d_latent), lambda i: (0, 0)),
            pl.BlockSpec((1, 1, tn), lambda i: (i, 0, 0)),
            pl.BlockSpec((1, k), lambda i: (0, 0)),
            pl.BlockSpec((1, k), lambda i: (0, 0)),
        ],
        out_specs=[
            pl.BlockSpec((d_latent, tn), lambda i: (0, i)),
            pl.BlockSpec((tn, k), lambda i: (i, 0)),
            pl.BlockSpec((tn, k), lambda i: (i, 0)),
            pl.BlockSpec((tn,), lambda i: (i,)),
        ],
        out_shape=[
            jax.ShapeDtypeStruct((d_latent, n), jnp.float32),
            jax.ShapeDtypeStruct((n, k), jnp.float32),
            jax.ShapeDtypeStruct((n, k), jnp.float32),
            jax.ShapeDtypeStruct((n,), jnp.int32),
        ],
    )(zt, codebook, zsq3, c_sq, iota_f)

    return zqt.T, q, q2, bmu
